# baseline (speedup 1.0000x reference)
import jax
import jax.numpy as jnp
from jax import lax
from jax.experimental import pallas as pl
from jax.experimental.pallas import tpu as pltpu
from jax.experimental.pallas import tpu_sc as plsc

E = 320000
N = 10000
EMB = 128
NR = 16
NOUT = 12

NC = 2
NS = 16
NW = NC * NS

CHUNK = 128
NCHUNKS = E // CHUNK
N_PAD = 10240
ROWS_PER_SUB = N_PAD // NS
KMAX = (NCHUNKS + NW - 1) // NW
KMAX2 = KMAX + (KMAX % 2)


SLICES = (51200, 89600, 89600, 89600)
M = len(SLICES)
EBLOCK = 12800


def _edge_body(rbft_ref, x_ref, w_ref, o_ref):
    g = lax.dot_general(rbft_ref[...], w_ref[...],
                        (((0,), (0,)), ((), ())),
                        preferred_element_type=jnp.float32)
    o_ref[...] = g * x_ref[...]


def _edge_body_chained(rbft_ref, x_ref, w_ref, prev_ref, o_ref):
    del prev_ref
    _edge_body(rbft_ref, x_ref, w_ref, o_ref)


def _edge_stage(x, rbf_t, w_rbf, start, nedges, prev=None, block=EBLOCK):
    grid = (nedges // block,)
    off = start // block
    in_specs = [
        pl.BlockSpec((NR, block), lambda i: (0, i + off)),
        pl.BlockSpec((block, EMB), lambda i: (i + off, 0)),
        pl.BlockSpec((NR, EMB), lambda i: (0, 0)),
    ]
    args = [rbf_t, x, w_rbf]
    body = _edge_body
    if prev is not None:
        in_specs.append(pl.BlockSpec((8, EMB), lambda i: (0, 0)))
        args.append(prev)
        body = _edge_body_chained
    return pl.pallas_call(
        body,
        grid=grid,
        in_specs=in_specs,
        out_specs=pl.BlockSpec((block, EMB), lambda i: (i, 0)),
        out_shape=jax.ShapeDtypeStruct((nedges, EMB), jnp.float32),
    )(*args)


def _make_scatter_body(schunks, skp):
    def _scatter_body(h_hbm, idxp_hbm, out_hbm,
                      idx_p, rows_v0, rows_v1, acc_sh, sem0, sem1):
        c = lax.axis_index("c")
        s = lax.axis_index("s")
        wid = s * NC + c

        pltpu.sync_copy(idxp_hbm.at[wid], idx_p)

        zvec = jnp.zeros((16,), jnp.float32)

        @pl.loop(0, CHUNK)
        def _(r):
            for lane in range(EMB // 16):
                rows_v0[r, pl.ds(lane * 16, 16)] = zvec

        for rep in range(ROWS_PER_SUB // CHUNK):
            pltpu.sync_copy(
                rows_v0,
                acc_sh.at[pl.ds(s * ROWS_PER_SUB + rep * CHUNK, CHUNK)],
            )
        plsc.subcore_barrier()

        def load(rows_v, sem, t):
            chunk = wid + NW * t

            @pl.when(chunk < schunks)
            def _():
                pltpu.async_copy(h_hbm.at[chunk], rows_v, sem)

        def scat(rows_v, sem, t):
            chunk = wid + NW * t

            @pl.when(chunk < schunks)
            def _():
                pltpu.make_async_copy(h_hbm.at[chunk], rows_v, sem).wait()
                pltpu.sync_copy(rows_v, acc_sh.at[idx_p.at[t]], add=True)

        load(rows_v0, sem0, 0)

        @pl.loop(0, skp, step=2)
        def _(t):
            load(rows_v1, sem1, t + 1)
            scat(rows_v0, sem0, t)
            load(rows_v0, sem0, t + 2)
            scat(rows_v1, sem1, t + 1)

        plsc.subcore_barrier()
        pltpu.sync_copy(
            acc_sh.at[pl.ds(s * ROWS_PER_SUB, ROWS_PER_SUB)],
            out_hbm.at[c, pl.ds(s * ROWS_PER_SUB, ROWS_PER_SUB)],
        )

    return _scatter_body


def _scatter_stage(h, idx_panel):
    schunks = h.shape[0] // CHUNK
    skp = idx_panel.shape[1]
    h3 = h.reshape(schunks, CHUNK, EMB)
    mesh = plsc.VectorSubcoreMesh(core_axis_name="c", subcore_axis_name="s")
    kern = pl.kernel(
        _make_scatter_body(schunks, skp),
        out_type=jax.ShapeDtypeStruct((NC, N_PAD, EMB), jnp.float32),
        mesh=mesh,
        scratch_types=[
            pltpu.VMEM((skp, 128), jnp.int32),
            pltpu.VMEM((CHUNK, EMB), jnp.float32),
            pltpu.VMEM((CHUNK, EMB), jnp.float32),
            pltpu.VMEM_SHARED((N_PAD, EMB), jnp.float32),
            pltpu.SemaphoreType.DMA,
            pltpu.SemaphoreType.DMA,
        ],
    )
    return kern(h3, idx_panel)


def _idx_panel(idx_flat, base_chunk, schunks):
    skmax = (schunks + NW - 1) // NW
    skp = skmax + (skmax % 2)
    a = lax.dynamic_slice(idx_flat, (base_chunk * CHUNK,), (schunks * CHUNK,))
    a = a.reshape(schunks, CHUNK)
    a = jnp.pad(a, ((0, skp * NW - schunks), (0, 0)))
    return a.reshape(skp, NW, CHUNK).transpose(1, 0, 2)


def _mlp_body(p0_ref, p1_ref, p2_ref, p3_ref,
              w1_ref, b1_ref, w2_ref, b2_ref, w3_ref, b3_ref,
              wo_ref, bo_ref, o_ref):
    y = ((p0_ref[0] + p0_ref[1]) + (p1_ref[0] + p1_ref[1])) + \
        ((p2_ref[0] + p2_ref[1]) + (p3_ref[0] + p3_ref[1]))
    y = jnp.dot(y, w1_ref[...], preferred_element_type=jnp.float32) + b1_ref[...]
    y = y * jax.nn.sigmoid(y)
    y = jnp.dot(y, w2_ref[...], preferred_element_type=jnp.float32) + b2_ref[...]
    y = y * jax.nn.sigmoid(y)
    y = jnp.dot(y, w3_ref[...], preferred_element_type=jnp.float32) + b3_ref[...]
    y = y * jax.nn.sigmoid(y)
    o_ref[...] = jnp.dot(y, wo_ref[...], preferred_element_type=jnp.float32) + bo_ref[...]


def _mlp_stage(parts, W1, b1, W2, b2, W3, b3, W_out, b_out, block=1000):
    wo = jnp.zeros((EMB, EMB), jnp.float32).at[:, :NOUT].set(W_out)
    bo = jnp.zeros((1, EMB), jnp.float32).at[0, :NOUT].set(b_out)
    grid = (N // block,)

    def full(shape):
        return pl.BlockSpec(shape, lambda i: tuple(0 for _ in shape))

    part_spec = pl.BlockSpec((NC, block, EMB), lambda i: (0, i, 0))
    out = pl.pallas_call(
        _mlp_body,
        grid=grid,
        in_specs=[
            part_spec, part_spec, part_spec, part_spec,
            full((EMB, EMB)), full((1, EMB)),
            full((EMB, EMB)), full((1, EMB)),
            full((EMB, EMB)), full((1, EMB)),
            full((EMB, EMB)), full((1, EMB)),
        ],
        out_specs=pl.BlockSpec((block, EMB), lambda i: (i, 0)),
        out_shape=jax.ShapeDtypeStruct((N, EMB), jnp.float32),
    )(*parts, W1, b1.reshape(1, EMB), W2, b2.reshape(1, EMB),
      W3, b3.reshape(1, EMB), wo, bo)
    return out[:, :NOUT]


def kernel(x, rbf, idnb_i, W_rbf, W1, b1, W2, b2, W3, b3, W_out, b_out):
    rbf_t = rbf.T
    idx_flat = idnb_i.astype(jnp.int32)
    parts = []
    start = 0
    h_m = None
    for nedges in SLICES:
        panel = _idx_panel(idx_flat, start // CHUNK, nedges // CHUNK)
        h_m = _edge_stage(x, rbf_t, W_rbf, start, nedges, prev=h_m)
        parts.append(_scatter_stage(h_m, panel))
        start += nedges
    return _mlp_stage(parts, W1, b1, W2, b2, W3, b3, W_out, b_out)


# speedup vs baseline: 1.0238x; 1.0238x over previous
import jax
import jax.numpy as jnp
from jax import lax
from jax.experimental import pallas as pl
from jax.experimental.pallas import tpu as pltpu
from jax.experimental.pallas import tpu_sc as plsc

E = 320000
N = 10000
EMB = 128
NR = 16
NOUT = 12

NC = 2
NS = 16
NW = NC * NS

CHUNK = 128
NCHUNKS = E // CHUNK
N_PAD = 10240
ROWS_PER_SUB = N_PAD // NS
KMAX = (NCHUNKS + NW - 1) // NW
KMAX2 = KMAX + (KMAX % 2)


SLICES = (25600, 89600, 102400, 102400)
M = len(SLICES)
EBLOCK = 12800


def _edge_body(rbft_ref, x_ref, w_ref, o_ref):
    g = lax.dot_general(rbft_ref[...], w_ref[...],
                        (((0,), (0,)), ((), ())),
                        preferred_element_type=jnp.float32)
    o_ref[...] = g * x_ref[...]


def _edge_body_chained(rbft_ref, x_ref, w_ref, prev_ref, o_ref):
    del prev_ref
    _edge_body(rbft_ref, x_ref, w_ref, o_ref)


def _edge_stage(x, rbf_t, w_rbf, start, nedges, prev=None, block=EBLOCK):
    grid = (nedges // block,)
    off = start // block
    in_specs = [
        pl.BlockSpec((NR, block), lambda i: (0, i + off)),
        pl.BlockSpec((block, EMB), lambda i: (i + off, 0)),
        pl.BlockSpec((NR, EMB), lambda i: (0, 0)),
    ]
    args = [rbf_t, x, w_rbf]
    body = _edge_body
    if prev is not None:
        in_specs.append(pl.BlockSpec((8, EMB), lambda i: (0, 0)))
        args.append(prev)
        body = _edge_body_chained
    return pl.pallas_call(
        body,
        grid=grid,
        in_specs=in_specs,
        out_specs=pl.BlockSpec((block, EMB), lambda i: (i, 0)),
        out_shape=jax.ShapeDtypeStruct((nedges, EMB), jnp.float32),
    )(*args)


def _make_scatter_body(schunks, skp):
    def _scatter_body(h_hbm, idxp_hbm, out_hbm,
                      idx_p, rows_v0, rows_v1, acc_sh, sem0, sem1):
        c = lax.axis_index("c")
        s = lax.axis_index("s")
        wid = s * NC + c

        pltpu.sync_copy(idxp_hbm.at[wid], idx_p)

        zvec = jnp.zeros((16,), jnp.float32)

        @pl.loop(0, CHUNK)
        def _(r):
            for lane in range(EMB // 16):
                rows_v0[r, pl.ds(lane * 16, 16)] = zvec

        for rep in range(ROWS_PER_SUB // CHUNK):
            pltpu.sync_copy(
                rows_v0,
                acc_sh.at[pl.ds(s * ROWS_PER_SUB + rep * CHUNK, CHUNK)],
            )
        plsc.subcore_barrier()

        def load(rows_v, sem, t):
            chunk = wid + NW * t

            @pl.when(chunk < schunks)
            def _():
                pltpu.async_copy(h_hbm.at[chunk], rows_v, sem)

        def scat(rows_v, sem, t):
            chunk = wid + NW * t

            @pl.when(chunk < schunks)
            def _():
                pltpu.make_async_copy(h_hbm.at[chunk], rows_v, sem).wait()
                pltpu.sync_copy(rows_v, acc_sh.at[idx_p.at[t]], add=True)

        load(rows_v0, sem0, 0)

        @pl.loop(0, skp, step=2)
        def _(t):
            load(rows_v1, sem1, t + 1)
            scat(rows_v0, sem0, t)
            load(rows_v0, sem0, t + 2)
            scat(rows_v1, sem1, t + 1)

        plsc.subcore_barrier()
        pltpu.sync_copy(
            acc_sh.at[pl.ds(s * ROWS_PER_SUB, ROWS_PER_SUB)],
            out_hbm.at[c, pl.ds(s * ROWS_PER_SUB, ROWS_PER_SUB)],
        )

    return _scatter_body


def _scatter_stage(h, idx_panel):
    schunks = h.shape[0] // CHUNK
    skp = idx_panel.shape[1]
    h3 = h.reshape(schunks, CHUNK, EMB)
    mesh = plsc.VectorSubcoreMesh(core_axis_name="c", subcore_axis_name="s")
    kern = pl.kernel(
        _make_scatter_body(schunks, skp),
        out_type=jax.ShapeDtypeStruct((NC, N_PAD, EMB), jnp.float32),
        mesh=mesh,
        scratch_types=[
            pltpu.VMEM((skp, 128), jnp.int32),
            pltpu.VMEM((CHUNK, EMB), jnp.float32),
            pltpu.VMEM((CHUNK, EMB), jnp.float32),
            pltpu.VMEM_SHARED((N_PAD, EMB), jnp.float32),
            pltpu.SemaphoreType.DMA,
            pltpu.SemaphoreType.DMA,
        ],
    )
    return kern(h3, idx_panel)


def _idx_panel(idx_flat, base_chunk, schunks):
    skmax = (schunks + NW - 1) // NW
    skp = skmax + (skmax % 2)
    a = lax.dynamic_slice(idx_flat, (base_chunk * CHUNK,), (schunks * CHUNK,))
    a = a.reshape(schunks, CHUNK)
    a = jnp.pad(a, ((0, skp * NW - schunks), (0, 0)))
    return a.reshape(skp, NW, CHUNK).transpose(1, 0, 2)


def _mlp_body(p0_ref, p1_ref, p2_ref, p3_ref,
              w1_ref, b1_ref, w2_ref, b2_ref, w3_ref, b3_ref,
              wo_ref, bo_ref, o_ref):
    y = ((p0_ref[0] + p0_ref[1]) + (p1_ref[0] + p1_ref[1])) + \
        ((p2_ref[0] + p2_ref[1]) + (p3_ref[0] + p3_ref[1]))
    y = jnp.dot(y, w1_ref[...], preferred_element_type=jnp.float32) + b1_ref[...]
    y = y * jax.nn.sigmoid(y)
    y = jnp.dot(y, w2_ref[...], preferred_element_type=jnp.float32) + b2_ref[...]
    y = y * jax.nn.sigmoid(y)
    y = jnp.dot(y, w3_ref[...], preferred_element_type=jnp.float32) + b3_ref[...]
    y = y * jax.nn.sigmoid(y)
    o_ref[...] = jnp.dot(y, wo_ref[...], preferred_element_type=jnp.float32) + bo_ref[...]


def _mlp_stage(parts, W1, b1, W2, b2, W3, b3, W_out, b_out, block=2000):
    wo = jnp.zeros((EMB, EMB), jnp.float32).at[:, :NOUT].set(W_out)
    bo = jnp.zeros((1, EMB), jnp.float32).at[0, :NOUT].set(b_out)
    grid = (N // block,)

    def full(shape):
        return pl.BlockSpec(shape, lambda i: tuple(0 for _ in shape))

    part_spec = pl.BlockSpec((NC, block, EMB), lambda i: (0, i, 0))
    out = pl.pallas_call(
        _mlp_body,
        grid=grid,
        in_specs=[
            part_spec, part_spec, part_spec, part_spec,
            full((EMB, EMB)), full((1, EMB)),
            full((EMB, EMB)), full((1, EMB)),
            full((EMB, EMB)), full((1, EMB)),
            full((EMB, EMB)), full((1, EMB)),
        ],
        out_specs=pl.BlockSpec((block, EMB), lambda i: (i, 0)),
        out_shape=jax.ShapeDtypeStruct((N, EMB), jnp.float32),
    )(*parts, W1, b1.reshape(1, EMB), W2, b2.reshape(1, EMB),
      W3, b3.reshape(1, EMB), wo, bo)
    return out[:, :NOUT]


def kernel(x, rbf, idnb_i, W_rbf, W1, b1, W2, b2, W3, b3, W_out, b_out):
    rbf_t = rbf.T
    idx_flat = idnb_i.astype(jnp.int32)
    parts = []
    start = 0
    h_m = None
    for nedges in SLICES:
        panel = _idx_panel(idx_flat, start // CHUNK, nedges // CHUNK)
        h_m = _edge_stage(x, rbf_t, W_rbf, start, nedges, prev=h_m)
        parts.append(_scatter_stage(h_m, panel))
        start += nedges
    return _mlp_stage(parts, W1, b1, W2, b2, W3, b3, W_out, b_out)


# speedup vs baseline: 1.0363x; 1.0121x over previous
import jax
import jax.numpy as jnp
from jax import lax
from jax.experimental import pallas as pl
from jax.experimental.pallas import tpu as pltpu
from jax.experimental.pallas import tpu_sc as plsc

E = 320000
N = 10000
EMB = 128
NR = 16
NOUT = 12

NC = 2
NS = 16
NW = NC * NS

CHUNK = 128
NCHUNKS = E // CHUNK
N_PAD = 10240
ROWS_PER_SUB = N_PAD // NS
KMAX = (NCHUNKS + NW - 1) // NW
KMAX2 = KMAX + (KMAX % 2)


SLICES = (25600, 89600, 102400, 102400)
M = len(SLICES)
EBLOCK = 12800


def _edge_body(rbft_ref, x_ref, w_ref, o_ref):
    g = lax.dot_general(rbft_ref[...], w_ref[...],
                        (((0,), (0,)), ((), ())),
                        preferred_element_type=jnp.float32)
    o_ref[...] = g * x_ref[...]


def _edge_body_chained(rbft_ref, x_ref, w_ref, prev_ref, o_ref):
    del prev_ref
    _edge_body(rbft_ref, x_ref, w_ref, o_ref)


def _edge_stage(x, rbf_t, w_rbf, start, nedges, prev=None, block=EBLOCK):
    grid = (nedges // block,)
    off = start // block
    in_specs = [
        pl.BlockSpec((NR, block), lambda i: (0, i + off)),
        pl.BlockSpec((block, EMB), lambda i: (i + off, 0)),
        pl.BlockSpec((NR, EMB), lambda i: (0, 0)),
    ]
    args = [rbf_t, x, w_rbf]
    body = _edge_body
    if prev is not None:
        in_specs.append(pl.BlockSpec((8, EMB), lambda i: (0, 0)))
        args.append(prev)
        body = _edge_body_chained
    return pl.pallas_call(
        body,
        grid=grid,
        in_specs=in_specs,
        out_specs=pl.BlockSpec((block, EMB), lambda i: (i, 0)),
        out_shape=jax.ShapeDtypeStruct((nedges, EMB), jnp.float32),
    )(*args)


def _make_scatter_body(schunks, skp):
    def _scatter_body(h_hbm, idxp_hbm, out_hbm,
                      idx_p, rows_v0, rows_v1, acc_sh, sem0, sem1):
        c = lax.axis_index("c")
        s = lax.axis_index("s")
        wid = s * NC + c

        pltpu.sync_copy(idxp_hbm.at[wid], idx_p)

        zvec = jnp.zeros((16,), jnp.float32)

        @pl.loop(0, CHUNK)
        def _(r):
            for lane in range(EMB // 16):
                rows_v0[r, pl.ds(lane * 16, 16)] = zvec

        for rep in range(ROWS_PER_SUB // CHUNK):
            pltpu.sync_copy(
                rows_v0,
                acc_sh.at[pl.ds(s * ROWS_PER_SUB + rep * CHUNK, CHUNK)],
            )
        plsc.subcore_barrier()

        def load(rows_v, sem, t):
            chunk = wid + NW * t

            @pl.when(chunk < schunks)
            def _():
                pltpu.async_copy(h_hbm.at[chunk], rows_v, sem)

        def scat(rows_v, sem, t):
            chunk = wid + NW * t

            @pl.when(chunk < schunks)
            def _():
                pltpu.make_async_copy(h_hbm.at[chunk], rows_v, sem).wait()
                pltpu.sync_copy(rows_v, acc_sh.at[idx_p.at[t]], add=True)

        load(rows_v0, sem0, 0)

        @pl.loop(0, skp, step=2)
        def _(t):
            load(rows_v1, sem1, t + 1)
            scat(rows_v0, sem0, t)
            load(rows_v0, sem0, t + 2)
            scat(rows_v1, sem1, t + 1)

        plsc.subcore_barrier()
        pltpu.sync_copy(
            acc_sh.at[pl.ds(s * ROWS_PER_SUB, ROWS_PER_SUB)],
            out_hbm.at[c, pl.ds(s * ROWS_PER_SUB, ROWS_PER_SUB)],
        )

    return _scatter_body


def _scatter_stage(h, idx_panel):
    schunks = h.shape[0] // CHUNK
    skp = idx_panel.shape[1]
    h3 = h.reshape(schunks, CHUNK, EMB)
    mesh = plsc.VectorSubcoreMesh(core_axis_name="c", subcore_axis_name="s")
    kern = pl.kernel(
        _make_scatter_body(schunks, skp),
        out_type=jax.ShapeDtypeStruct((NC, N_PAD, EMB), jnp.float32),
        mesh=mesh,
        scratch_types=[
            pltpu.VMEM((skp, 128), jnp.int32),
            pltpu.VMEM((CHUNK, EMB), jnp.float32),
            pltpu.VMEM((CHUNK, EMB), jnp.float32),
            pltpu.VMEM_SHARED((N_PAD, EMB), jnp.float32),
            pltpu.SemaphoreType.DMA,
            pltpu.SemaphoreType.DMA,
        ],
    )
    return kern(h3, idx_panel)


def _idx_panel(idx_flat, base_chunk, schunks):
    skmax = (schunks + NW - 1) // NW
    skp = skmax + (skmax % 2)
    a = lax.dynamic_slice(idx_flat, (base_chunk * CHUNK,), (schunks * CHUNK,))
    a = a.reshape(schunks, CHUNK)
    a = jnp.pad(a, ((0, skp * NW - schunks), (0, 0)))
    return a.reshape(skp, NW, CHUNK).transpose(1, 0, 2)


def _presum_body(p0_ref, p1_ref, p2_ref, o_ref):
    o_ref[...] = ((p0_ref[0] + p0_ref[1]) + (p1_ref[0] + p1_ref[1])) + \
                 (p2_ref[0] + p2_ref[1])


def _presum_stage(parts, block=2048):
    grid = (N_PAD // block,)
    part_spec = pl.BlockSpec((NC, block, EMB), lambda i: (0, i, 0))
    return pl.pallas_call(
        _presum_body,
        grid=grid,
        in_specs=[part_spec, part_spec, part_spec],
        out_specs=pl.BlockSpec((block, EMB), lambda i: (i, 0)),
        out_shape=jax.ShapeDtypeStruct((N_PAD, EMB), jnp.float32),
    )(*parts)


def _mlp_body(pre_ref, p3_ref,
              w1_ref, b1_ref, w2_ref, b2_ref, w3_ref, b3_ref,
              wo_ref, bo_ref, o_ref):
    y = pre_ref[...] + (p3_ref[0] + p3_ref[1])
    y = jnp.dot(y, w1_ref[...], preferred_element_type=jnp.float32) + b1_ref[...]
    y = y * jax.nn.sigmoid(y)
    y = jnp.dot(y, w2_ref[...], preferred_element_type=jnp.float32) + b2_ref[...]
    y = y * jax.nn.sigmoid(y)
    y = jnp.dot(y, w3_ref[...], preferred_element_type=jnp.float32) + b3_ref[...]
    y = y * jax.nn.sigmoid(y)
    o_ref[...] = jnp.dot(y, wo_ref[...], preferred_element_type=jnp.float32) + bo_ref[...]


def _mlp_stage(parts, W1, b1, W2, b2, W3, b3, W_out, b_out, block=2000):
    wo = jnp.zeros((EMB, EMB), jnp.float32).at[:, :NOUT].set(W_out)
    bo = jnp.zeros((1, EMB), jnp.float32).at[0, :NOUT].set(b_out)
    grid = (N // block,)

    def full(shape):
        return pl.BlockSpec(shape, lambda i: tuple(0 for _ in shape))

    pre, p3 = parts
    part_spec = pl.BlockSpec((NC, block, EMB), lambda i: (0, i, 0))
    out = pl.pallas_call(
        _mlp_body,
        grid=grid,
        in_specs=[
            pl.BlockSpec((block, EMB), lambda i: (i, 0)), part_spec,
            full((EMB, EMB)), full((1, EMB)),
            full((EMB, EMB)), full((1, EMB)),
            full((EMB, EMB)), full((1, EMB)),
            full((EMB, EMB)), full((1, EMB)),
        ],
        out_specs=pl.BlockSpec((block, EMB), lambda i: (i, 0)),
        out_shape=jax.ShapeDtypeStruct((N, EMB), jnp.float32),
    )(*parts, W1, b1.reshape(1, EMB), W2, b2.reshape(1, EMB),
      W3, b3.reshape(1, EMB), wo, bo)
    return out[:, :NOUT]


def kernel(x, rbf, idnb_i, W_rbf, W1, b1, W2, b2, W3, b3, W_out, b_out):
    rbf_t = rbf.T
    idx_flat = idnb_i.astype(jnp.int32)
    parts = []
    start = 0
    h_m = None
    for nedges in SLICES:
        panel = _idx_panel(idx_flat, start // CHUNK, nedges // CHUNK)
        h_m = _edge_stage(x, rbf_t, W_rbf, start, nedges, prev=h_m)
        parts.append(_scatter_stage(h_m, panel))
        start += nedges
    pre = _presum_stage(parts[:3])
    return _mlp_stage([pre, parts[3]], W1, b1, W2, b2, W3, b3, W_out, b_out)


# speedup vs baseline: 1.0367x; 1.0004x over previous
import jax
import jax.numpy as jnp
from jax import lax
from jax.experimental import pallas as pl
from jax.experimental.pallas import tpu as pltpu
from jax.experimental.pallas import tpu_sc as plsc

E = 320000
N = 10000
EMB = 128
NR = 16
NOUT = 12

NC = 2
NS = 16
NW = NC * NS

CHUNK = 128
N_PAD = 10240
ROWS_PER_SUB = N_PAD // NS

SLICES = (25600, 89600, 102400, 102400)
M = len(SLICES)
EBLOCK = 12800


def _edge_body(rbft_ref, x_ref, w_ref, o_ref):
    g = lax.dot_general(rbft_ref[...], w_ref[...],
                        (((0,), (0,)), ((), ())),
                        preferred_element_type=jnp.float32)
    o_ref[...] = g * x_ref[...]


def _edge_body_chained(rbft_ref, x_ref, w_ref, prev_ref, o_ref):
    del prev_ref
    _edge_body(rbft_ref, x_ref, w_ref, o_ref)


def _edge_stage(x, rbf_t, w_rbf, start, nedges, prev=None, block=EBLOCK):
    grid = (nedges // block,)
    off = start // block
    in_specs = [
        pl.BlockSpec((NR, block), lambda i: (0, i + off)),
        pl.BlockSpec((block, EMB), lambda i: (i + off, 0)),
        pl.BlockSpec((NR, EMB), lambda i: (0, 0)),
    ]
    args = [rbf_t, x, w_rbf]
    body = _edge_body
    if prev is not None:
        in_specs.append(pl.BlockSpec((8, EMB), lambda i: (0, 0)))
        args.append(prev)
        body = _edge_body_chained
    return pl.pallas_call(
        body,
        grid=grid,
        in_specs=in_specs,
        out_specs=pl.BlockSpec((block, EMB), lambda i: (i, 0)),
        out_shape=jax.ShapeDtypeStruct((nedges, EMB), jnp.float32),
    )(*args)


def _make_scatter_body(schunks, skp):
    def _scatter_body(h_hbm, idxp_hbm, out_hbm,
                      idx_p, rows_v0, rows_v1, acc_sh, sem0, sem1):
        c = lax.axis_index("c")
        s = lax.axis_index("s")
        wid = s * NC + c

        pltpu.sync_copy(idxp_hbm.at[wid], idx_p)

        zvec = jnp.zeros((16,), jnp.float32)

        @pl.loop(0, CHUNK)
        def _(r):
            for lane in range(EMB // 16):
                rows_v0[r, pl.ds(lane * 16, 16)] = zvec

        for rep in range(ROWS_PER_SUB // CHUNK):
            pltpu.sync_copy(
                rows_v0,
                acc_sh.at[pl.ds(s * ROWS_PER_SUB + rep * CHUNK, CHUNK)],
            )
        plsc.subcore_barrier()

        def load(rows_v, sem, t):
            chunk = wid + NW * t

            @pl.when(chunk < schunks)
            def _():
                pltpu.async_copy(h_hbm.at[chunk], rows_v, sem)

        def scat(rows_v, sem, t):
            chunk = wid + NW * t

            @pl.when(chunk < schunks)
            def _():
                pltpu.make_async_copy(h_hbm.at[chunk], rows_v, sem).wait()
                pltpu.sync_copy(rows_v, acc_sh.at[idx_p.at[t]], add=True)

        load(rows_v0, sem0, 0)

        @pl.loop(0, skp, step=2)
        def _(t):
            load(rows_v1, sem1, t + 1)
            scat(rows_v0, sem0, t)
            load(rows_v0, sem0, t + 2)
            scat(rows_v1, sem1, t + 1)

        plsc.subcore_barrier()
        pltpu.sync_copy(
            acc_sh.at[pl.ds(s * ROWS_PER_SUB, ROWS_PER_SUB)],
            out_hbm.at[c, pl.ds(s * ROWS_PER_SUB, ROWS_PER_SUB)],
        )

    return _scatter_body


def _scatter_stage(h, idx_panel):
    schunks = h.shape[0] // CHUNK
    skp = idx_panel.shape[1]
    h3 = h.reshape(schunks, CHUNK, EMB)
    mesh = plsc.VectorSubcoreMesh(core_axis_name="c", subcore_axis_name="s")
    kern = pl.kernel(
        _make_scatter_body(schunks, skp),
        out_type=jax.ShapeDtypeStruct((NC, N_PAD, EMB), jnp.float32),
        mesh=mesh,
        scratch_types=[
            pltpu.VMEM((skp, 128), jnp.int32),
            pltpu.VMEM((CHUNK, EMB), jnp.float32),
            pltpu.VMEM((CHUNK, EMB), jnp.float32),
            pltpu.VMEM_SHARED((N_PAD, EMB), jnp.float32),
            pltpu.SemaphoreType.DMA,
            pltpu.SemaphoreType.DMA,
        ],
    )
    return kern(h3, idx_panel)


def _idx_panel(idx_flat, base_chunk, schunks):
    skmax = (schunks + NW - 1) // NW
    skp = skmax + (skmax % 2)
    a = lax.dynamic_slice(idx_flat, (base_chunk * CHUNK,), (schunks * CHUNK,))
    a = a.reshape(schunks, CHUNK)
    a = jnp.pad(a, ((0, skp * NW - schunks), (0, 0)))
    return a.reshape(skp, NW, CHUNK).transpose(1, 0, 2)


def _presum_body(p0_ref, p1_ref, p2_ref, o_ref):
    o_ref[...] = ((p0_ref[0] + p0_ref[1]) + (p1_ref[0] + p1_ref[1])) + \
                 (p2_ref[0] + p2_ref[1])


def _presum_stage(parts, block=2048):
    grid = (N_PAD // block,)
    part_spec = pl.BlockSpec((NC, block, EMB), lambda i: (0, i, 0))
    return pl.pallas_call(
        _presum_body,
        grid=grid,
        in_specs=[part_spec, part_spec, part_spec],
        out_specs=pl.BlockSpec((block, EMB), lambda i: (i, 0)),
        out_shape=jax.ShapeDtypeStruct((N_PAD, EMB), jnp.float32),
    )(*parts)


def _mlp_body(pre_ref, p3_ref,
              w1_ref, b1_ref, w2_ref, b2_ref, w3_ref, b3_ref,
              wo_ref, bo_ref, o_ref):
    y = pre_ref[...] + (p3_ref[0] + p3_ref[1])
    y = jnp.dot(y, w1_ref[...], preferred_element_type=jnp.float32) + b1_ref[...]
    y = y * jax.nn.sigmoid(y)
    y = jnp.dot(y, w2_ref[...], preferred_element_type=jnp.float32) + b2_ref[...]
    y = y * jax.nn.sigmoid(y)
    y = jnp.dot(y, w3_ref[...], preferred_element_type=jnp.float32) + b3_ref[...]
    y = y * jax.nn.sigmoid(y)
    o_ref[...] = jnp.dot(y, wo_ref[...], preferred_element_type=jnp.float32) + bo_ref[...]


def _mlp_stage(parts, W1, b1, W2, b2, W3, b3, W_out, b_out, block=2000):
    wo = jnp.zeros((EMB, EMB), jnp.float32).at[:, :NOUT].set(W_out)
    bo = jnp.zeros((1, EMB), jnp.float32).at[0, :NOUT].set(b_out)
    grid = (N // block,)

    def full(shape):
        return pl.BlockSpec(shape, lambda i: tuple(0 for _ in shape))

    pre, p3 = parts
    part_spec = pl.BlockSpec((NC, block, EMB), lambda i: (0, i, 0))
    out = pl.pallas_call(
        _mlp_body,
        grid=grid,
        in_specs=[
            pl.BlockSpec((block, EMB), lambda i: (i, 0)), part_spec,
            full((EMB, EMB)), full((1, EMB)),
            full((EMB, EMB)), full((1, EMB)),
            full((EMB, EMB)), full((1, EMB)),
            full((EMB, EMB)), full((1, EMB)),
        ],
        out_specs=pl.BlockSpec((block, EMB), lambda i: (i, 0)),
        out_shape=jax.ShapeDtypeStruct((N, EMB), jnp.float32),
    )(*parts, W1, b1.reshape(1, EMB), W2, b2.reshape(1, EMB),
      W3, b3.reshape(1, EMB), wo, bo)
    return out[:, :NOUT]


def kernel(x, rbf, idnb_i, W_rbf, W1, b1, W2, b2, W3, b3, W_out, b_out):
    rbf_t = rbf.T
    idx_flat = idnb_i.astype(jnp.int32)
    parts = []
    start = 0
    h_m = None
    for nedges in SLICES:
        panel = _idx_panel(idx_flat, start // CHUNK, nedges // CHUNK)
        h_m = _edge_stage(x, rbf_t, W_rbf, start, nedges, prev=h_m)
        parts.append(_scatter_stage(h_m, panel))
        start += nedges
    pre = _presum_stage(parts[:3])
    return _mlp_stage([pre, parts[3]], W1, b1, W2, b2, W3, b3, W_out, b_out)
